# Initial kernel scaffold; baseline (speedup 1.0000x reference)
import jax
import jax.numpy as jnp
from jax.experimental import pallas as pl


def kernel(z, edge_index, distances, mol_ids, boltz_weights, conf_ids, atom_embed, conv_Wf1, conv_bf1, conv_Wf2, conv_bf2, conv_Win, conv_bin, conv_Wo1, conv_bo1, conv_Wo2, conv_bo2, mol_W1, mol_b1, mol_W2, mol_b2, read_W1, read_b1, read_W2, read_b2):
    raise NotImplementedError("write your pallas kernel here")


# speedup vs baseline: 1.9231x; 1.9231x over previous
import functools

import jax
import jax.numpy as jnp
from jax import lax
from jax.experimental import pallas as pl
from jax.experimental.pallas import tpu as pltpu
from jax.experimental.pallas import tpu_sc as plsc

N_ATOMS = 10000
N_EDGES = 320000
N_CONFS = 500
N_SPECIES = 100
D = 128
NG = 32
NCONV = 3
CUTOFF = 5.0
LN2 = 0.6931471805599453

NC = 2
NS = 16
NW = NC * NS
CH = 128
E_PAD = 327680
PER_W = E_PAD // NW
N_CHUNK = PER_W // CH
A_PAD = 10240
ROWS_PER_TILE = A_PAD // NS

HI = lax.Precision.HIGHEST


def _ssp(x):
    return jnp.maximum(x, 0.0) + jnp.log1p(jnp.exp(-jnp.abs(x))) - LN2


B_E = 2048


def _wmat_body(dist_ref, wf1_ref, bf1_ref, wf2_ref, bf2_ref, out_ref):
    j = pl.program_id(0)
    d = dist_ref[...]
    width = CUTOFF / (NG - 1)
    offs = lax.broadcasted_iota(jnp.int32, (B_E, NG), 1).astype(
        jnp.float32) * width
    t = (d - offs) * (1.0 / width)
    e = jnp.exp(-0.5 * t * t)
    t1 = _ssp(jnp.dot(e, wf1_ref[...], preferred_element_type=jnp.float32)
              + bf1_ref[...])
    wm = jnp.dot(t1, wf2_ref[...], preferred_element_type=jnp.float32) \
        + bf2_ref[...]
    rid = lax.broadcasted_iota(jnp.int32, (B_E, D), 0) + j * B_E
    out_ref[...] = jnp.where(rid < N_EDGES, wm, 0.0)


def _wmat(dist2, wf1, bf1, wf2, bf2):
    nb = E_PAD // B_E
    return pl.pallas_call(
        _wmat_body,
        grid=(nb,),
        in_specs=[
            pl.BlockSpec((B_E, 1), lambda j: (j, 0)),
            pl.BlockSpec((NG, D), lambda j: (0, 0)),
            pl.BlockSpec((1, D), lambda j: (0, 0)),
            pl.BlockSpec((D, D), lambda j: (0, 0)),
            pl.BlockSpec((1, D), lambda j: (0, 0)),
        ],
        out_specs=pl.BlockSpec((B_E, D), lambda j: (j, 0)),
        out_shape=jax.ShapeDtypeStruct((E_PAD, D), jnp.float32),
    )(dist2, wf1, bf1, wf2, bf2)


def _embed_body(z_ref, tab_ref, out_ref):
    oh = (z_ref[...] == lax.broadcasted_iota(jnp.int32, (A_PAD, 128), 1))
    out_ref[...] = jnp.dot(oh.astype(jnp.float32), tab_ref[...],
                           preferred_element_type=jnp.float32, precision=HI)


def _embed(z2, tab_pad):
    return pl.pallas_call(
        _embed_body,
        out_shape=jax.ShapeDtypeStruct((A_PAD, D), jnp.float32),
    )(z2, tab_pad)


def _hmat_body(r_ref, w_ref, b_ref, out_ref):
    out_ref[...] = jnp.dot(r_ref[...], w_ref[...],
                           preferred_element_type=jnp.float32) + b_ref[...]


def _hmat(r, w, b):
    return pl.pallas_call(
        _hmat_body,
        out_shape=jax.ShapeDtypeStruct((A_PAD, D), jnp.float32),
    )(r, w, b)


def _update_body(p_ref, r_ref, w1_ref, b1_ref, w2_ref, b2_ref, out_ref):
    agg = p_ref[0] + p_ref[1]
    t = _ssp(jnp.dot(agg, w1_ref[...], preferred_element_type=jnp.float32)
             + b1_ref[...])
    out_ref[...] = r_ref[...] + jnp.dot(
        t, w2_ref[...], preferred_element_type=jnp.float32) + b2_ref[...]


def _update(p, r, w1, b1, w2, b2):
    return pl.pallas_call(
        _update_body,
        out_shape=jax.ShapeDtypeStruct((A_PAD, D), jnp.float32),
    )(p, r, w1, b1, w2, b2)


def _edge_body(h_hbm, wmat_hbm, src_hbm, dst_hbm, out_hbm,
               src_v, dst_v, rows_v, wm_v, agg_sh, sem):
    cid = lax.axis_index("c")
    sid = lax.axis_index("s")
    wid = sid * NC + cid

    def _zrow(i, _):
        for j in range(D // 16):
            rows_v[i, pl.ds(j * 16, 16)] = jnp.zeros((16,), jnp.float32)
        return 0

    lax.fori_loop(0, CH, _zrow, 0)
    base_r = sid * ROWS_PER_TILE
    for k in range(ROWS_PER_TILE // CH):
        pltpu.sync_copy(rows_v, agg_sh.at[pl.ds(base_r + k * CH, CH)])
    plsc.subcore_barrier()

    def _chunk(t, _):
        base_e = wid * PER_W + t * CH
        pltpu.sync_copy(src_hbm.at[pl.ds(base_e, CH)], src_v)
        pltpu.sync_copy(dst_hbm.at[pl.ds(base_e, CH)], dst_v)
        pltpu.async_copy(h_hbm.at[src_v], rows_v, sem).wait()
        pltpu.sync_copy(wmat_hbm.at[pl.ds(base_e, CH)], wm_v)

        def _mul(i, _):
            for j in range(D // 16):
                sl = pl.ds(j * 16, 16)
                rows_v[i, sl] = rows_v[i, sl] * wm_v[i, sl]
            return 0

        lax.fori_loop(0, CH, _mul, 0)
        pltpu.sync_copy(rows_v, agg_sh.at[dst_v], add=True)
        return 0

    lax.fori_loop(0, N_CHUNK, _chunk, 0)
    plsc.subcore_barrier()

    for k in range(ROWS_PER_TILE // CH):
        sl = pl.ds(base_r + k * CH, CH)
        pltpu.sync_copy(agg_sh.at[sl], out_hbm.at[cid].at[sl])


def _edge_op(h, wmat, src, dst):
    mesh = plsc.VectorSubcoreMesh(core_axis_name="c", subcore_axis_name="s",
                                  num_cores=NC, num_subcores=NS)
    return pl.kernel(
        _edge_body,
        out_type=jax.ShapeDtypeStruct((NC, A_PAD, D), jnp.float32),
        mesh=mesh,
        scratch_types=[
            pltpu.VMEM((CH,), jnp.int32),
            pltpu.VMEM((CH,), jnp.int32),
            pltpu.VMEM((CH, D), jnp.float32),
            pltpu.VMEM((CH, D), jnp.float32),
            pltpu.VMEM_SHARED((A_PAD, D), jnp.float32),
            pltpu.SemaphoreType.DMA,
        ],
    )(h, wmat, src, dst)


B_A = 2000
NCP = 512
NSP = 128


def _readout_body(r_ref, mol_ref, boltz_ref, cid_ref,
                  mw1_ref, mb1_ref, mw2_ref, mb2_ref,
                  rw1_ref, rb1_ref, rw2_ref, rb2_ref,
                  out_ref, acc_ref):
    j = pl.program_id(0)

    @pl.when(j == 0)
    def _():
        acc_ref[...] = jnp.zeros_like(acc_ref)

    oh = (mol_ref[...] == lax.broadcasted_iota(jnp.int32, (B_A, NCP),
                                               1).astype(jnp.float32))
    acc_ref[...] += lax.dot_general(
        oh.astype(jnp.float32), r_ref[...],
        (((0,), (0,)), ((), ())), precision=HI,
        preferred_element_type=jnp.float32)

    @pl.when(j == (N_ATOMS // B_A) - 1)
    def _():
        conf = acc_ref[...]
        conf = jnp.dot(_ssp(jnp.dot(conf, mw1_ref[...],
                                    preferred_element_type=jnp.float32)
                            + mb1_ref[...]),
                       mw2_ref[...], preferred_element_type=jnp.float32) \
            + mb2_ref[...]
        wtd = conf * boltz_ref[...]
        oh2 = (cid_ref[...] == lax.broadcasted_iota(jnp.int32, (NCP, NSP),
                                                    1).astype(jnp.float32))
        spec = lax.dot_general(
            oh2.astype(jnp.float32), wtd,
            (((0,), (0,)), ((), ())), precision=HI,
            preferred_element_type=jnp.float32)
        z1 = _ssp(jnp.dot(spec, rw1_ref[...],
                          preferred_element_type=jnp.float32) + rb1_ref[...])
        out_ref[...] = jax.nn.sigmoid(
            jnp.dot(z1, rw2_ref[...], preferred_element_type=jnp.float32)
            + rb2_ref[...])


def _readout(r, mol_f, boltz_pad, cid_f, mw1, mb1, mw2, mb2,
             rw1, rb1, rw2, rb2):
    nb = N_ATOMS // B_A
    c = lambda j: (0, 0)
    return pl.pallas_call(
        _readout_body,
        grid=(nb,),
        in_specs=[
            pl.BlockSpec((B_A, D), lambda j: (j, 0)),
            pl.BlockSpec((B_A, 1), lambda j: (j, 0)),
            pl.BlockSpec((NCP, 1), c),
            pl.BlockSpec((NCP, 1), c),
            pl.BlockSpec((D, D), c),
            pl.BlockSpec((1, D), c),
            pl.BlockSpec((D, D), c),
            pl.BlockSpec((1, D), c),
            pl.BlockSpec((D, D // 2), c),
            pl.BlockSpec((1, D // 2), c),
            pl.BlockSpec((D // 2, 1), c),
            pl.BlockSpec((1, 1), c),
        ],
        out_specs=pl.BlockSpec((NSP, 1), c),
        out_shape=jax.ShapeDtypeStruct((NSP, 1), jnp.float32),
        scratch_shapes=[pltpu.VMEM((NCP, D), jnp.float32)],
    )(r, mol_f, boltz_pad, cid_f, mw1, mb1, mw2, mb2, rw1, rb1, rw2, rb2)


def kernel(z, edge_index, distances, mol_ids, boltz_weights, conf_ids,
           atom_embed, conv_Wf1, conv_bf1, conv_Wf2, conv_bf2, conv_Win,
           conv_bin, conv_Wo1, conv_bo1, conv_Wo2, conv_bo2, mol_W1, mol_b1,
           mol_W2, mol_b2, read_W1, read_b1, read_W2, read_b2):
    pad_e = E_PAD - N_EDGES
    src = jnp.pad(edge_index[0].astype(jnp.int32), (0, pad_e))
    dst = jnp.pad(edge_index[1].astype(jnp.int32), (0, pad_e))
    dist2 = jnp.pad(distances, (0, pad_e)).reshape(E_PAD, 1)
    z2 = jnp.pad(z.astype(jnp.int32), (0, A_PAD - N_ATOMS),
                 constant_values=127).reshape(A_PAD, 1)
    tab_pad = jnp.pad(atom_embed, ((0, 128 - N_SPECIES), (0, 0)))
    mol_f = mol_ids.astype(jnp.float32).reshape(N_ATOMS, 1)
    boltz_pad = jnp.pad(boltz_weights, (0, NCP - N_CONFS)).reshape(NCP, 1)
    cid_f = jnp.pad(conf_ids.astype(jnp.float32), (0, NCP - N_CONFS),
                    constant_values=float(NSP - 1)).reshape(NCP, 1)
    b2 = lambda b: b.reshape(1, -1)

    r = _embed(z2, tab_pad)
    for i in range(NCONV):
        wm = _wmat(dist2, conv_Wf1[i], b2(conv_bf1[i]),
                   conv_Wf2[i], b2(conv_bf2[i]))
        h = _hmat(r, conv_Win[i], b2(conv_bin[i]))
        p = _edge_op(h, wm, src, dst)
        r = _update(p, r, conv_Wo1[i], b2(conv_bo1[i]),
                    conv_Wo2[i], b2(conv_bo2[i]))

    out = _readout(r, mol_f, boltz_pad, cid_f,
                   mol_W1, b2(mol_b1), mol_W2, b2(mol_b2),
                   read_W1, b2(read_b1), read_W2, b2(read_b2))
    return out[:N_SPECIES]


# speedup vs baseline: 2.5097x; 1.3050x over previous
import functools

import jax
import jax.numpy as jnp
from jax import lax
from jax.experimental import pallas as pl
from jax.experimental.pallas import tpu as pltpu
from jax.experimental.pallas import tpu_sc as plsc

N_ATOMS = 10000
N_EDGES = 320000
N_CONFS = 500
N_SPECIES = 100
D = 128
NG = 32
NCONV = 3
CUTOFF = 5.0
LN2 = 0.6931471805599453

NC = 2
NS = 16
NW = NC * NS
CH = 64
E_PAD = 327680
PER_W = E_PAD // NW
N_CHUNK = PER_W // CH
A_PAD = 10240
ROWS_PER_TILE = A_PAD // NS

HI = lax.Precision.HIGHEST


def _ssp(x):
    return jnp.maximum(x, 0.0) + jnp.log1p(jnp.exp(-jnp.abs(x))) - LN2


B_E = 2048


def _wmat_body(dist_ref, wf1_ref, bf1_ref, wf2_ref, bf2_ref, out_ref):
    j = pl.program_id(0)
    d = dist_ref[...]
    width = CUTOFF / (NG - 1)
    offs = lax.broadcasted_iota(jnp.int32, (B_E, NG), 1).astype(
        jnp.float32) * width
    t = (d - offs) * (1.0 / width)
    e = jnp.exp(-0.5 * t * t)
    t1 = _ssp(jnp.dot(e, wf1_ref[...], preferred_element_type=jnp.float32)
              + bf1_ref[...])
    wm = jnp.dot(t1, wf2_ref[...], preferred_element_type=jnp.float32) \
        + bf2_ref[...]
    rid = lax.broadcasted_iota(jnp.int32, (B_E, D), 0) + j * B_E
    out_ref[...] = jnp.where(rid < N_EDGES, wm, 0.0)


def _wmat(dist2, wf1, bf1, wf2, bf2):
    nb = E_PAD // B_E
    return pl.pallas_call(
        _wmat_body,
        grid=(nb,),
        in_specs=[
            pl.BlockSpec((B_E, 1), lambda j: (j, 0)),
            pl.BlockSpec((NG, D), lambda j: (0, 0)),
            pl.BlockSpec((1, D), lambda j: (0, 0)),
            pl.BlockSpec((D, D), lambda j: (0, 0)),
            pl.BlockSpec((1, D), lambda j: (0, 0)),
        ],
        out_specs=pl.BlockSpec((B_E, D), lambda j: (j, 0)),
        out_shape=jax.ShapeDtypeStruct((E_PAD, D), jnp.float32),
    )(dist2, wf1, bf1, wf2, bf2)


def _embed_body(z_ref, tab_ref, out_ref):
    oh = (z_ref[...] == lax.broadcasted_iota(jnp.int32, (A_PAD, 128), 1))
    out_ref[...] = jnp.dot(oh.astype(jnp.float32), tab_ref[...],
                           preferred_element_type=jnp.float32, precision=HI)


def _embed(z2, tab_pad):
    return pl.pallas_call(
        _embed_body,
        out_shape=jax.ShapeDtypeStruct((A_PAD, D), jnp.float32),
    )(z2, tab_pad)


def _hmat_body(r_ref, w_ref, b_ref, out_ref):
    out_ref[...] = jnp.dot(r_ref[...], w_ref[...],
                           preferred_element_type=jnp.float32) + b_ref[...]


def _hmat(r, w, b):
    return pl.pallas_call(
        _hmat_body,
        out_shape=jax.ShapeDtypeStruct((A_PAD, D), jnp.float32),
    )(r, w, b)


def _update_body(p_ref, r_ref, w1_ref, b1_ref, w2_ref, b2_ref, out_ref):
    agg = p_ref[0] + p_ref[1]
    t = _ssp(jnp.dot(agg, w1_ref[...], preferred_element_type=jnp.float32)
             + b1_ref[...])
    out_ref[...] = r_ref[...] + jnp.dot(
        t, w2_ref[...], preferred_element_type=jnp.float32) + b2_ref[...]


def _update(p, r, w1, b1, w2, b2):
    return pl.pallas_call(
        _update_body,
        out_shape=jax.ShapeDtypeStruct((A_PAD, D), jnp.float32),
    )(p, r, w1, b1, w2, b2)


def _edge_body(h_hbm, wmat_hbm, src_hbm, dst_hbm, out_hbm,
               src0, src1, wm0, wm1, rows0, rows1,
               dst0, dst1, dst2, dst3, agg_sh,
               semi0, semi1, semg0, semg1, sems0, sems1):
    cid = lax.axis_index("c")
    sid = lax.axis_index("s")
    wid = sid * NC + cid

    SRC = (src0, src1)
    WM = (wm0, wm1)
    ROWS = (rows0, rows1)
    DST = (dst0, dst1, dst2, dst3)
    SEMI = (semi0, semi1)
    SEMG = (semg0, semg1)
    SEMS = (sems0, sems1)

    def issue_in(t, b2, b4):
        b = wid * PER_W + t * CH
        pltpu.async_copy(src_hbm.at[pl.ds(b, CH)], SRC[b2], SEMI[b2])
        pltpu.async_copy(dst_hbm.at[pl.ds(b, CH)], DST[b4], SEMI[b2])
        pltpu.async_copy(wmat_hbm.at[pl.ds(b, CH)], WM[b2], SEMI[b2])

    def wait_in(b2, b4):
        pltpu.make_async_copy(src_hbm.at[pl.ds(0, CH)], SRC[b2],
                              SEMI[b2]).wait()
        pltpu.make_async_copy(dst_hbm.at[pl.ds(0, CH)], DST[b4],
                              SEMI[b2]).wait()
        pltpu.make_async_copy(wmat_hbm.at[pl.ds(0, CH)], WM[b2],
                              SEMI[b2]).wait()

    def start_gather(b2):
        pltpu.async_copy(h_hbm.at[SRC[b2]], ROWS[b2], SEMG[b2])

    def wait_gather(b2):
        pltpu.make_async_copy(h_hbm.at[SRC[b2]], ROWS[b2], SEMG[b2]).wait()

    def start_scatter(b2, b4):
        pltpu.async_copy(ROWS[b2], agg_sh.at[DST[b4]], SEMS[b2], add=True)

    def wait_scatter(b2, b4):
        pltpu.make_async_copy(ROWS[b2], agg_sh.at[DST[b4]], SEMS[b2]).wait()

    def mul(b2):
        r, w = ROWS[b2], WM[b2]

        def _m(i, _):
            for j in range(D // 16):
                sl = pl.ds(j * 16, 16)
                r[i, sl] = r[i, sl] * w[i, sl]
            return 0

        lax.fori_loop(0, CH, _m, 0)

    def _zrow(i, _):
        for j in range(D // 16):
            rows0[i, pl.ds(j * 16, 16)] = jnp.zeros((16,), jnp.float32)
        return 0

    lax.fori_loop(0, CH, _zrow, 0)
    base_r = sid * ROWS_PER_TILE
    for k in range(ROWS_PER_TILE // CH):
        pltpu.sync_copy(rows0, agg_sh.at[pl.ds(base_r + k * CH, CH)])
    plsc.subcore_barrier()

    def steady(t, k, first=False, last=False):
        tb, nb = k % 2, (k + 1) % 2
        wait_in(nb, (k + 1) % 4)
        if not first:
            wait_scatter(nb, (k + 3) % 4)
        start_gather(nb)
        wait_gather(tb)
        mul(tb)
        if not last:
            issue_in(t + 2, tb, (k + 2) % 4)
        start_scatter(tb, k % 4)

    issue_in(0, 0, 0)
    wait_in(0, 0)
    start_gather(0)
    issue_in(1, 1, 1)
    steady(0, 0, first=True)
    steady(1, 1)
    def _loop(u, _):
        t = 2 + 4 * u
        for k in range(4):
            steady(t + k, (2 + k))
        return 0

    lax.fori_loop(0, (N_CHUNK - 4) // 4, _loop, 0)
    steady(N_CHUNK - 2, 2, last=True)
    wait_gather(1)
    mul(1)
    start_scatter(1, 3)
    wait_scatter(0, 2)
    wait_scatter(1, 3)
    plsc.subcore_barrier()

    for k in range(ROWS_PER_TILE // CH):
        sl = pl.ds(base_r + k * CH, CH)
        pltpu.sync_copy(agg_sh.at[sl], out_hbm.at[cid].at[sl])


def _edge_op(h, wmat, src, dst):
    mesh = plsc.VectorSubcoreMesh(core_axis_name="c", subcore_axis_name="s",
                                  num_cores=NC, num_subcores=NS)
    return pl.kernel(
        _edge_body,
        out_type=jax.ShapeDtypeStruct((NC, A_PAD, D), jnp.float32),
        mesh=mesh,
        scratch_types=[
            pltpu.VMEM((CH,), jnp.int32),
            pltpu.VMEM((CH,), jnp.int32),
            pltpu.VMEM((CH, D), jnp.float32),
            pltpu.VMEM((CH, D), jnp.float32),
            pltpu.VMEM((CH, D), jnp.float32),
            pltpu.VMEM((CH, D), jnp.float32),
            pltpu.VMEM((CH,), jnp.int32),
            pltpu.VMEM((CH,), jnp.int32),
            pltpu.VMEM((CH,), jnp.int32),
            pltpu.VMEM((CH,), jnp.int32),
            pltpu.VMEM_SHARED((A_PAD, D), jnp.float32),
            pltpu.SemaphoreType.DMA,
            pltpu.SemaphoreType.DMA,
            pltpu.SemaphoreType.DMA,
            pltpu.SemaphoreType.DMA,
            pltpu.SemaphoreType.DMA,
            pltpu.SemaphoreType.DMA,
        ],
    )(h, wmat, src, dst)


B_A = 2000
NCP = 512
NSP = 128


def _readout_body(r_ref, mol_ref, boltz_ref, cid_ref,
                  mw1_ref, mb1_ref, mw2_ref, mb2_ref,
                  rw1_ref, rb1_ref, rw2_ref, rb2_ref,
                  out_ref, acc_ref):
    j = pl.program_id(0)

    @pl.when(j == 0)
    def _():
        acc_ref[...] = jnp.zeros_like(acc_ref)

    oh = (mol_ref[...] == lax.broadcasted_iota(jnp.int32, (B_A, NCP),
                                               1).astype(jnp.float32))
    acc_ref[...] += lax.dot_general(
        oh.astype(jnp.float32), r_ref[...],
        (((0,), (0,)), ((), ())), precision=HI,
        preferred_element_type=jnp.float32)

    @pl.when(j == (N_ATOMS // B_A) - 1)
    def _():
        conf = acc_ref[...]
        conf = jnp.dot(_ssp(jnp.dot(conf, mw1_ref[...],
                                    preferred_element_type=jnp.float32)
                            + mb1_ref[...]),
                       mw2_ref[...], preferred_element_type=jnp.float32) \
            + mb2_ref[...]
        wtd = conf * boltz_ref[...]
        oh2 = (cid_ref[...] == lax.broadcasted_iota(jnp.int32, (NCP, NSP),
                                                    1).astype(jnp.float32))
        spec = lax.dot_general(
            oh2.astype(jnp.float32), wtd,
            (((0,), (0,)), ((), ())), precision=HI,
            preferred_element_type=jnp.float32)
        z1 = _ssp(jnp.dot(spec, rw1_ref[...],
                          preferred_element_type=jnp.float32) + rb1_ref[...])
        out_ref[...] = jax.nn.sigmoid(
            jnp.dot(z1, rw2_ref[...], preferred_element_type=jnp.float32)
            + rb2_ref[...])


def _readout(r, mol_f, boltz_pad, cid_f, mw1, mb1, mw2, mb2,
             rw1, rb1, rw2, rb2):
    nb = N_ATOMS // B_A
    c = lambda j: (0, 0)
    return pl.pallas_call(
        _readout_body,
        grid=(nb,),
        in_specs=[
            pl.BlockSpec((B_A, D), lambda j: (j, 0)),
            pl.BlockSpec((B_A, 1), lambda j: (j, 0)),
            pl.BlockSpec((NCP, 1), c),
            pl.BlockSpec((NCP, 1), c),
            pl.BlockSpec((D, D), c),
            pl.BlockSpec((1, D), c),
            pl.BlockSpec((D, D), c),
            pl.BlockSpec((1, D), c),
            pl.BlockSpec((D, D // 2), c),
            pl.BlockSpec((1, D // 2), c),
            pl.BlockSpec((D // 2, 1), c),
            pl.BlockSpec((1, 1), c),
        ],
        out_specs=pl.BlockSpec((NSP, 1), c),
        out_shape=jax.ShapeDtypeStruct((NSP, 1), jnp.float32),
        scratch_shapes=[pltpu.VMEM((NCP, D), jnp.float32)],
    )(r, mol_f, boltz_pad, cid_f, mw1, mb1, mw2, mb2, rw1, rb1, rw2, rb2)


def kernel(z, edge_index, distances, mol_ids, boltz_weights, conf_ids,
           atom_embed, conv_Wf1, conv_bf1, conv_Wf2, conv_bf2, conv_Win,
           conv_bin, conv_Wo1, conv_bo1, conv_Wo2, conv_bo2, mol_W1, mol_b1,
           mol_W2, mol_b2, read_W1, read_b1, read_W2, read_b2):
    pad_e = E_PAD - N_EDGES
    src = jnp.pad(edge_index[0].astype(jnp.int32), (0, pad_e))
    dst = jnp.pad(edge_index[1].astype(jnp.int32), (0, pad_e))
    dist2 = jnp.pad(distances, (0, pad_e)).reshape(E_PAD, 1)
    z2 = jnp.pad(z.astype(jnp.int32), (0, A_PAD - N_ATOMS),
                 constant_values=127).reshape(A_PAD, 1)
    tab_pad = jnp.pad(atom_embed, ((0, 128 - N_SPECIES), (0, 0)))
    mol_f = mol_ids.astype(jnp.float32).reshape(N_ATOMS, 1)
    boltz_pad = jnp.pad(boltz_weights, (0, NCP - N_CONFS)).reshape(NCP, 1)
    cid_f = jnp.pad(conf_ids.astype(jnp.float32), (0, NCP - N_CONFS),
                    constant_values=float(NSP - 1)).reshape(NCP, 1)
    b2 = lambda b: b.reshape(1, -1)

    r = _embed(z2, tab_pad)
    for i in range(NCONV):
        wm = _wmat(dist2, conv_Wf1[i], b2(conv_bf1[i]),
                   conv_Wf2[i], b2(conv_bf2[i]))
        h = _hmat(r, conv_Win[i], b2(conv_bin[i]))
        p = _edge_op(h, wm, src, dst)
        r = _update(p, r, conv_Wo1[i], b2(conv_bo1[i]),
                    conv_Wo2[i], b2(conv_bo2[i]))

    out = _readout(r, mol_f, boltz_pad, cid_f,
                   mol_W1, b2(mol_b1), mol_W2, b2(mol_b2),
                   read_W1, b2(read_b1), read_W2, b2(read_b2))
    return out[:N_SPECIES]
